# Initial kernel scaffold; baseline (speedup 1.0000x reference)
import jax
import jax.numpy as jnp
from jax.experimental import pallas as pl


def kernel(words_embeddings, token_type_ids, position_table, token_type_table, ln_gamma, ln_beta):
    raise NotImplementedError("write your pallas kernel here")


# speedup vs baseline: 3.1429x; 3.1429x over previous
import jax
import jax.numpy as jnp
from jax.experimental import pallas as pl
from jax.experimental.pallas import tpu as pltpu

_EPS = 1e-12
_BS = 512


def _emb_ln_kernel(ids_ref, words_ref, pos_ref, tt_ref, gamma_ref, beta_ref,
                   out_ref):
    x = words_ref[0] + pos_ref[...]
    idf = ids_ref[0].astype(jnp.float32)
    tt0 = tt_ref[0][None, :]
    diff = (tt_ref[1] - tt_ref[0])[None, :]
    x = x + tt0 + idf * diff
    h = x.shape[-1]
    mu = jnp.sum(x, axis=-1, keepdims=True) * (1.0 / h)
    xc = x - mu
    var = jnp.sum(xc * xc, axis=-1, keepdims=True) * (1.0 / h)
    y = xc * jax.lax.rsqrt(var + _EPS)
    out_ref[0] = y * gamma_ref[...] + beta_ref[...]


def kernel(words_embeddings, token_type_ids, position_table,
           token_type_table, ln_gamma, ln_beta):
    b, s, h = words_embeddings.shape
    bs = min(_BS, s)
    ids3 = token_type_ids.astype(jnp.int32).reshape(b, s, 1)
    gamma2 = ln_gamma.reshape(1, h)
    beta2 = ln_beta.reshape(1, h)

    grid = (s // bs, b)
    return pl.pallas_call(
        _emb_ln_kernel,
        grid=grid,
        in_specs=[
            pl.BlockSpec((1, bs, 1), lambda i, j: (j, i, 0)),
            pl.BlockSpec((1, bs, h), lambda i, j: (j, i, 0)),
            pl.BlockSpec((bs, h), lambda i, j: (i, 0)),
            pl.BlockSpec(token_type_table.shape, lambda i, j: (0, 0)),
            pl.BlockSpec((1, h), lambda i, j: (0, 0)),
            pl.BlockSpec((1, h), lambda i, j: (0, 0)),
        ],
        out_specs=pl.BlockSpec((1, bs, h), lambda i, j: (j, i, 0)),
        out_shape=jax.ShapeDtypeStruct((b, s, h), jnp.float32),
        compiler_params=pltpu.CompilerParams(
            dimension_semantics=("parallel", "parallel"),
        ),
    )(ids3, words_embeddings, position_table, token_type_table, gamma2,
      beta2)


# speedup vs baseline: 3.4984x; 1.1131x over previous
import jax
import jax.numpy as jnp
from jax.experimental import pallas as pl
from jax.experimental.pallas import tpu as pltpu

_EPS = 1e-12
_BS = 1024


def _emb_ln_kernel(ids_ref, words_ref, pos_ref, tt_ref, gamma_ref, beta_ref,
                   out_ref):
    x = words_ref[0] + pos_ref[...]
    idf = ids_ref[0].astype(jnp.float32)
    tt0 = tt_ref[0][None, :]
    diff = (tt_ref[1] - tt_ref[0])[None, :]
    x = x + tt0 + idf * diff
    h = x.shape[-1]
    mu = jnp.sum(x, axis=-1, keepdims=True) * (1.0 / h)
    xc = x - mu
    var = jnp.sum(xc * xc, axis=-1, keepdims=True) * (1.0 / h)
    y = xc * jax.lax.rsqrt(var + _EPS)
    out_ref[0] = y * gamma_ref[...] + beta_ref[...]


def kernel(words_embeddings, token_type_ids, position_table,
           token_type_table, ln_gamma, ln_beta):
    b, s, h = words_embeddings.shape
    bs = min(_BS, s)
    ids3 = token_type_ids.astype(jnp.int32).reshape(b, s, 1)
    gamma2 = ln_gamma.reshape(1, h)
    beta2 = ln_beta.reshape(1, h)

    grid = (s // bs, b)
    return pl.pallas_call(
        _emb_ln_kernel,
        grid=grid,
        in_specs=[
            pl.BlockSpec((1, bs, 1), lambda i, j: (j, i, 0)),
            pl.BlockSpec((1, bs, h), lambda i, j: (j, i, 0)),
            pl.BlockSpec((bs, h), lambda i, j: (i, 0)),
            pl.BlockSpec(token_type_table.shape, lambda i, j: (0, 0)),
            pl.BlockSpec((1, h), lambda i, j: (0, 0)),
            pl.BlockSpec((1, h), lambda i, j: (0, 0)),
        ],
        out_specs=pl.BlockSpec((1, bs, h), lambda i, j: (j, i, 0)),
        out_shape=jax.ShapeDtypeStruct((b, s, h), jnp.float32),
        compiler_params=pltpu.CompilerParams(
            dimension_semantics=("parallel", "parallel"),
        ),
    )(ids3, words_embeddings, position_table, token_type_table, gamma2,
      beta2)


# speedup vs baseline: 3.8374x; 1.0969x over previous
import jax
import jax.numpy as jnp
from jax.experimental import pallas as pl
from jax.experimental.pallas import tpu as pltpu

_EPS = 1e-12
_BS = 512


def _emb_ln_kernel(ids_ref, words_ref, pos_ref, tt_ref, gamma_ref, beta_ref,
                   out_ref):
    x = words_ref[...] + pos_ref[...][None, :, :]
    idf = ids_ref[...].astype(jnp.float32)
    tt0 = tt_ref[0][None, None, :]
    diff = (tt_ref[1] - tt_ref[0])[None, None, :]
    x = x + tt0 + idf * diff
    h = x.shape[-1]
    mu = jnp.sum(x, axis=-1, keepdims=True) * (1.0 / h)
    xc = x - mu
    var = jnp.sum(xc * xc, axis=-1, keepdims=True) * (1.0 / h)
    y = xc * jax.lax.rsqrt(var + _EPS)
    out_ref[...] = y * gamma_ref[...][None, :, :] + beta_ref[...][None, :, :]


def kernel(words_embeddings, token_type_ids, position_table,
           token_type_table, ln_gamma, ln_beta):
    b, s, h = words_embeddings.shape
    bs = min(_BS, s)
    ids3 = token_type_ids.astype(jnp.int32).reshape(b, s, 1)
    gamma2 = ln_gamma.reshape(1, h)
    beta2 = ln_beta.reshape(1, h)

    grid = (s // bs,)
    return pl.pallas_call(
        _emb_ln_kernel,
        grid=grid,
        in_specs=[
            pl.BlockSpec((b, bs, 1), lambda i: (0, i, 0)),
            pl.BlockSpec((b, bs, h), lambda i: (0, i, 0)),
            pl.BlockSpec((bs, h), lambda i: (i, 0)),
            pl.BlockSpec(token_type_table.shape, lambda i: (0, 0)),
            pl.BlockSpec((1, h), lambda i: (0, 0)),
            pl.BlockSpec((1, h), lambda i: (0, 0)),
        ],
        out_specs=pl.BlockSpec((b, bs, h), lambda i: (0, i, 0)),
        out_shape=jax.ShapeDtypeStruct((b, s, h), jnp.float32),
        compiler_params=pltpu.CompilerParams(
            dimension_semantics=("parallel",),
        ),
    )(ids3, words_embeddings, position_table, token_type_table, gamma2,
      beta2)
